# baseline (speedup 1.0000x reference)
import functools

import jax
import jax.numpy as jnp
from jax import lax
from jax.experimental import pallas as pl
from jax.experimental.pallas import tpu as pltpu
from jax.experimental.pallas import tpu_sc as plsc

D_MODEL = 1024
BSZ = 4
SEQ = 4096
NC = 2
NS = 16
NW = NC * NS

B = BSZ * SEQ
B_PER_W = B // NW
CHUNK = 32
NCHUNKS = B_PER_W // CHUNK
WPR = SEQ // B_PER_W

_mesh = plsc.VectorSubcoreMesh(core_axis_name="c", subcore_axis_name="s")


@functools.partial(
    pl.kernel,
    mesh=_mesh,
    out_type=jax.ShapeDtypeStruct((BSZ, SEQ, D_MODEL), jnp.float32),
    scratch_types=[
        pltpu.VMEM((B_PER_W,), jnp.int32),
        pltpu.VMEM((3, CHUNK, D_MODEL), jnp.float32),
        pltpu.SemaphoreType.DMA,
        pltpu.SemaphoreType.DMA,
        pltpu.SemaphoreType.DMA,
        pltpu.SemaphoreType.DMA,
        pltpu.SemaphoreType.DMA,
        pltpu.SemaphoreType.DMA,
    ],
)
def _embed_gather(tok_hbm, table_hbm, out_hbm, idx_v, buf,
                  si0, si1, si2, so0, so1, so2):
    wid = lax.axis_index("s") * NC + lax.axis_index("c")
    b = wid // WPR
    s0 = (wid % WPR) * B_PER_W
    pltpu.sync_copy(tok_hbm.at[b, pl.ds(s0, B_PER_W)], idx_v)
    sin, sout = (si0, si1, si2), (so0, so1, so2)

    def gather(c):
        return pltpu.async_copy(
            table_hbm.at[idx_v.at[pl.ds(c * CHUNK, CHUNK)]],
            buf.at[c % 3], sin[c % 3])

    def put(c):
        return pltpu.async_copy(
            buf.at[c % 3], out_hbm.at[b, pl.ds(s0 + c * CHUNK, CHUNK)],
            sout[c % 3])

    g = [None] * NCHUNKS
    p = [None] * NCHUNKS
    g[0] = gather(0)
    g[1] = gather(1)
    for c in range(NCHUNKS):
        g[c].wait()
        p[c] = put(c)
        if c + 2 < NCHUNKS:
            if c >= 1:
                p[c - 1].wait()
            g[c + 2] = gather(c + 2)
    p[NCHUNKS - 2].wait()
    p[NCHUNKS - 1].wait()


def kernel(tokens, W_E):
    return _embed_gather(tokens, W_E)


# speedup vs baseline: 1.0301x; 1.0301x over previous
import functools

import jax
import jax.numpy as jnp
from jax import lax
from jax.experimental import pallas as pl
from jax.experimental.pallas import tpu as pltpu
from jax.experimental.pallas import tpu_sc as plsc

D_MODEL = 1024
BSZ = 4
SEQ = 4096
NC = 2
NS = 16
NW = NC * NS

B = BSZ * SEQ
B_PER_W = B // NW
CHUNK = 56
_CHUNKS = [CHUNK] * (B_PER_W // CHUNK) + (
    [B_PER_W % CHUNK] if B_PER_W % CHUNK else [])
_OFFS = [sum(_CHUNKS[:i]) for i in range(len(_CHUNKS))]
NCHUNKS = len(_CHUNKS)
WPR = SEQ // B_PER_W

_mesh = plsc.VectorSubcoreMesh(core_axis_name="c", subcore_axis_name="s")


@functools.partial(
    pl.kernel,
    mesh=_mesh,
    out_type=jax.ShapeDtypeStruct((BSZ, SEQ, D_MODEL), jnp.float32),
    scratch_types=[
        pltpu.VMEM((B_PER_W,), jnp.int32),
        pltpu.VMEM((2, CHUNK, D_MODEL), jnp.float32),
        pltpu.SemaphoreType.DMA,
        pltpu.SemaphoreType.DMA,
        pltpu.SemaphoreType.DMA,
        pltpu.SemaphoreType.DMA,
    ],
)
def _embed_gather(tok_hbm, table_hbm, out_hbm, idx_v, buf,
                  si0, si1, so0, so1):
    wid = lax.axis_index("s") * NC + lax.axis_index("c")
    b = wid // WPR
    s0 = (wid % WPR) * B_PER_W
    pltpu.sync_copy(tok_hbm.at[b, pl.ds(s0, B_PER_W)], idx_v)
    sin, sout = (si0, si1), (so0, so1)

    def gather(c):
        return pltpu.async_copy(
            table_hbm.at[idx_v.at[pl.ds(_OFFS[c], _CHUNKS[c])]],
            buf.at[c % 2, pl.ds(0, _CHUNKS[c])], sin[c % 2])

    def put(c):
        return pltpu.async_copy(
            buf.at[c % 2, pl.ds(0, _CHUNKS[c])],
            out_hbm.at[b, pl.ds(s0 + _OFFS[c], _CHUNKS[c])],
            sout[c % 2])

    g = [None] * NCHUNKS
    p = [None] * NCHUNKS
    g[0] = gather(0)
    for c in range(NCHUNKS):
        if c + 1 < NCHUNKS:
            if c >= 1:
                p[c - 1].wait()
            g[c + 1] = gather(c + 1)
        g[c].wait()
        p[c] = put(c)
    p[NCHUNKS - 2].wait()
    p[NCHUNKS - 1].wait()


def kernel(tokens, W_E):
    return _embed_gather(tokens, W_E)
